# baseline (speedup 1.0000x reference)
import functools

import jax
import jax.numpy as jnp
from jax import lax
from jax.experimental import pallas as pl
from jax.experimental.pallas import tpu as pltpu
from jax.experimental.pallas import tpu_sc as plsc

B = 16384
V = 89
D = 64
L = 16
NC = 2
NS = 16
NW = NC * NS
BPW = B // NW

_mesh = plsc.VectorSubcoreMesh(core_axis_name="c", subcore_axis_name="s")


@functools.partial(
    pl.kernel,
    out_type=jax.ShapeDtypeStruct((D, B), jnp.float32),
    mesh=_mesh,
    compiler_params=pltpu.CompilerParams(needs_layout_passes=False),
    scratch_types=[
        pltpu.VMEM((BPW,), jnp.int32),
        pltpu.VMEM((V * D,), jnp.float32),
        pltpu.VMEM((D * V,), jnp.float32),
        pltpu.VMEM((D, BPW), jnp.float32),
        pltpu.SemaphoreType.DMA,
        pltpu.SemaphoreType.DMA,
    ],
)
def _emb_transpose(x_hbm, table_hbm, out_hbm, idx_v, raw_v, tab_v, out_v,
                   in_sem, out_sem):
    wid = lax.axis_index("s") * NC + lax.axis_index("c")
    base = wid * BPW
    idx_dma = pltpu.async_copy(x_hbm.at[pl.ds(base, BPW)], idx_v, in_sem)
    tab_dma = pltpu.async_copy(table_hbm, raw_v, in_sem)
    tab_dma.wait()

    lane_v = jnp.arange(L, dtype=jnp.int32) * V

    @plsc.parallel_loop(0, V * D, D)
    def transpose(p):
        r = p // D
        for k in range(D // L):
            vals = raw_v[pl.ds(p + k * L, L)]
            plsc.store_scatter(tab_v, [lane_v + (k * L * V + r)], vals)

    idx_dma.wait()

    @plsc.parallel_loop(0, BPW, L)
    def group(b):
        xv = idx_v[pl.ds(b, L)]
        for d in range(D):
            out_v[d, pl.ds(b, L)] = plsc.load_gather(tab_v, [xv + (d * V)])

    pltpu.async_copy(out_v, out_hbm.at[:, pl.ds(base, BPW)], out_sem).wait()


def kernel(x, table):
    return _emb_transpose(x.astype(jnp.int32), table.reshape(V * D))


# speedup vs baseline: 1.0189x; 1.0189x over previous
import functools

import jax
import jax.numpy as jnp
from jax import lax
from jax.experimental import pallas as pl
from jax.experimental.pallas import tpu as pltpu
from jax.experimental.pallas import tpu_sc as plsc

B = 16384
V = 89
D = 64
L = 16
NC = 2
NS = 16
NW = NC * NS
BPW = B // NW

_mesh = plsc.VectorSubcoreMesh(core_axis_name="c", subcore_axis_name="s")


@functools.partial(
    pl.kernel,
    out_type=jax.ShapeDtypeStruct((D, B), jnp.float32),
    mesh=_mesh,
    compiler_params=pltpu.CompilerParams(needs_layout_passes=False),
    scratch_types=[
        pltpu.VMEM((BPW,), jnp.int32),
        pltpu.VMEM((D * V,), jnp.float32),
        pltpu.VMEM((D, BPW), jnp.float32),
        pltpu.SemaphoreType.DMA,
        pltpu.SemaphoreType.DMA,
    ],
)
def _emb_transpose(x_hbm, table_hbm, out_hbm, idx_v, tab_v, out_v, in_sem, out_sem):
    wid = lax.axis_index("s") * NC + lax.axis_index("c")
    base = wid * BPW
    idx_dma = pltpu.async_copy(x_hbm.at[pl.ds(base, BPW)], idx_v, in_sem)
    tab_dma = pltpu.async_copy(table_hbm, tab_v, in_sem)
    idx_dma.wait()
    tab_dma.wait()

    @plsc.parallel_loop(0, BPW, L)
    def group(b):
        xv = idx_v[pl.ds(b, L)]
        for d in range(D):
            out_v[d, pl.ds(b, L)] = plsc.load_gather(tab_v, [xv + (d * V)])

    pltpu.async_copy(out_v, out_hbm.at[:, pl.ds(base, BPW)], out_sem).wait()


def kernel(x, table):
    return _emb_transpose(x.astype(jnp.int32), table.T.reshape(D * V))
